# baseline (speedup 1.0000x reference)
import functools

import jax
import jax.numpy as jnp
from jax import lax
from jax.experimental import pallas as pl
from jax.experimental.pallas import tpu as pltpu
from jax.experimental.pallas import tpu_sc as plsc

N_NODES = 10000
NTILES = 32
TPSC = 16
K_EDGES = 80
NBUF = 8
ZR = 125
RPT = N_NODES // TPSC
WROWS = 632

_MBLK = 1000


def _make_mm_multi(n_parts):
    def _kern(*refs):
        x_ref = refs[0]
        ws = refs[1:1 + n_parts]
        outs = refs[1 + n_parts:]
        xv = x_ref[...]
        for w_ref, o_ref in zip(ws, outs):
            o_ref[...] = jnp.dot(xv, w_ref[...],
                                 preferred_element_type=jnp.float32)
    return _kern


def _matmul_multi(x, ws):
    m, k = x.shape
    n = ws[0].shape[1]
    return pl.pallas_call(
        _make_mm_multi(len(ws)),
        grid=(m // _MBLK,),
        in_specs=[pl.BlockSpec((_MBLK, k), lambda i: (i, 0))] +
                 [pl.BlockSpec((k, n), lambda i: (0, 0))] * len(ws),
        out_specs=[pl.BlockSpec((_MBLK, n), lambda i: (i, 0))] * len(ws),
        out_shape=[jax.ShapeDtypeStruct((m, n), jnp.float32)] * len(ws),
    )(x, *ws)


def _make_fuse(n_parts):
    def _kern(*refs):
        p0s = refs[0:n_parts]
        p1s = refs[n_parts:2 * n_parts]
        bs = refs[2 * n_parts:3 * n_parts]
        ws = refs[3 * n_parts:4 * n_parts]
        o_ref = refs[4 * n_parts]
        acc = None
        for p0, p1, b, w in zip(p0s, p1s, bs, ws):
            h = jnp.maximum(p0[...] + p1[...] + b[...], 0.0)
            t = jnp.dot(h, w[...], preferred_element_type=jnp.float32)
            acc = t if acc is None else acc + t
        o_ref[...] = acc
    return _kern


def _sum_relu_matmul(p0s, p1s, b1s, w2s):
    m, k = p0s[0].shape
    n = w2s[0].shape[1]
    np_ = len(p0s)
    blk = pl.BlockSpec((_MBLK, k), lambda i: (i, 0))
    return pl.pallas_call(
        _make_fuse(np_),
        grid=(m // _MBLK,),
        in_specs=[blk] * (2 * np_) +
                 [pl.BlockSpec((1, k), lambda i: (0, 0))] * np_ +
                 [pl.BlockSpec((k, n), lambda i: (0, 0))] * np_,
        out_specs=pl.BlockSpec((_MBLK, n), lambda i: (i, 0)),
        out_shape=jax.ShapeDtypeStruct((m, n), jnp.float32),
    )(*p0s, *p1s, *[b.reshape(1, k) for b in b1s], *w2s)


def _final_kernel(q0_ref, q1_ref, b2_ref, o_ref):
    o_ref[...] = q0_ref[...] + q1_ref[...] + b2_ref[...]


def _sum_bias(q0, q1, b2):
    m, n = q0.shape
    return pl.pallas_call(
        _final_kernel,
        grid=(m // _MBLK,),
        in_specs=[
            pl.BlockSpec((_MBLK, n), lambda i: (i, 0)),
            pl.BlockSpec((_MBLK, n), lambda i: (i, 0)),
            pl.BlockSpec((1, n), lambda i: (0, 0)),
        ],
        out_specs=pl.BlockSpec((_MBLK, n), lambda i: (i, 0)),
        out_shape=jax.ShapeDtypeStruct((m, n), jnp.float32),
    )(q0, q1, b2.reshape(1, n))


@functools.lru_cache(maxsize=None)
def _make_spmm(n_chunks: int, d: int, n_feats: int):
    mesh = plsc.VectorSubcoreMesh(core_axis_name="c", subcore_axis_name="s")
    groups = n_chunks // NBUF

    @functools.partial(
        pl.kernel,
        out_type=jax.ShapeDtypeStruct((n_feats * 2 * N_NODES, d), jnp.float32),
        mesh=mesh,
        compiler_params=pltpu.CompilerParams(use_tc_tiling_on_sc=False),
        scratch_types=(
            pltpu.VMEM((n_chunks, K_EDGES), jnp.int32),
            pltpu.VMEM((n_chunks, K_EDGES), jnp.int32),
            pltpu.VMEM((n_chunks, K_EDGES), jnp.float32),
            [pltpu.VMEM((K_EDGES, d), jnp.float32) for _ in range(NBUF)],
            pltpu.VMEM((ZR, d), jnp.float32),
            pltpu.VMEM_SHARED((N_NODES, d), jnp.float32),
            [pltpu.SemaphoreType.DMA for _ in range(NBUF)],
        ),
    )
    def spmm(*args):
        feats = args[:n_feats]
        (srcg, dstg, wg, out, srcb, dstb, wb, rows, zbuf, acc,
         gsems) = args[n_feats:]
        cid = lax.axis_index("c")
        sid = lax.axis_index("s")
        wid = cid * TPSC + sid
        r0 = sid * RPT

        pltpu.sync_copy(srcg.at[wid], srcb)
        pltpu.sync_copy(dstg.at[wid], dstb)
        pltpu.sync_copy(wg.at[wid], wb)

        def zrow(r, carry):
            for j in range(d // 16):
                zbuf[r, pl.ds(j * 16, 16)] = jnp.zeros((16,), jnp.float32)
            return carry
        lax.fori_loop(0, ZR, zrow, 0)

        for h in range(n_feats):
            feat = feats[h]

            for i in range(RPT // ZR):
                pltpu.sync_copy(zbuf, acc.at[pl.ds(r0 + i * ZR, ZR)])
            plsc.subcore_barrier()

            for b in range(NBUF):
                pltpu.async_copy(feat.at[srcb.at[b]], rows[b], gsems[b])

            def group(g, carry):
                for b in range(NBUF):
                    c = g * NBUF + b
                    pltpu.make_async_copy(
                        feat.at[srcb.at[c]], rows[b], gsems[b]).wait()

                    def scale_grp(gi, carry2):
                        e0 = gi * 16
                        wv = wb[c, pl.ds(e0, 16)]
                        for l in range(16):
                            wsc = wv[l]
                            for j in range(d // 16):
                                v = rows[b][e0 + l, pl.ds(j * 16, 16)]
                                rows[b][e0 + l, pl.ds(j * 16, 16)] = v * wsc
                        return carry2
                    lax.fori_loop(0, K_EDGES // 16, scale_grp, 0)

                    pltpu.sync_copy(rows[b], acc.at[dstb.at[c]], add=True)

                    @pl.when(g < groups - 1)
                    def _refill():
                        pltpu.async_copy(
                            feat.at[srcb.at[c + NBUF]], rows[b], gsems[b])
                return carry
            lax.fori_loop(0, groups, group, 0)

            plsc.subcore_barrier()
            last = N_NODES - 15 * WROWS
            r0w = sid * WROWS
            obase = (h * 2) * N_NODES + cid * N_NODES

            @pl.when(sid < 15)
            def _wmain():
                pltpu.sync_copy(acc.at[pl.ds(r0w, WROWS)],
                                out.at[pl.ds(obase + r0w, WROWS)])

            @pl.when(sid == 15)
            def _wlast():
                pltpu.sync_copy(acc.at[pl.ds(15 * WROWS, last)],
                                out.at[pl.ds(obase + 15 * WROWS, last)])

            plsc.subcore_barrier()

    return spmm


def _spmm_sc(feats, srcg, dstg, wg):
    n_chunks = srcg.shape[1]
    d = feats[0].shape[1]
    out = _make_spmm(n_chunks, d, len(feats))(*feats, srcg, dstg, wg)
    return [(out[(2 * h) * N_NODES:(2 * h + 1) * N_NODES],
             out[(2 * h + 1) * N_NODES:(2 * h + 2) * N_NODES])
            for h in range(len(feats))]


def kernel(x, edge_index, edge_weight, W1, b1, W2, b2):
    n_edges = edge_index.shape[1]
    n_chunks = -(-n_edges // (NTILES * K_EDGES))
    n_chunks = -(-n_chunks // NBUF) * NBUF
    pad_e = NTILES * n_chunks * K_EDGES - n_edges
    srcg = jnp.pad(edge_index[0].astype(jnp.int32), (0, pad_e)).reshape(
        NTILES, n_chunks, K_EDGES)
    dstg = jnp.pad(edge_index[1].astype(jnp.int32), (0, pad_e)).reshape(
        NTILES, n_chunks, K_EDGES)
    wg = jnp.pad(edge_weight, (0, pad_e)).reshape(NTILES, n_chunks, K_EDGES)

    dslab = 48
    nslab = 3
    nfeat = W1.shape[0]
    hpad = nslab * dslab - nfeat
    ncls = W2.shape[1]
    W1p = jnp.pad(W1, ((0, 0), (0, hpad)))
    b1p = jnp.pad(b1, (0, hpad))
    W2rp = jnp.pad(W2, ((0, hpad), (0, dslab - ncls)))
    b2p = jnp.pad(b2, (0, dslab - ncls))

    w1s = [W1p[:, i * dslab:(i + 1) * dslab] for i in range(nslab)]
    b1s = [b1p[i * dslab:(i + 1) * dslab] for i in range(nslab)]
    w2s = [W2rp[i * dslab:(i + 1) * dslab] for i in range(nslab)]

    sups = _matmul_multi(x, w1s)
    parts = _spmm_sc(sups, srcg, dstg, wg)
    support2 = _sum_relu_matmul([p[0] for p in parts], [p[1] for p in parts],
                                b1s, w2s)
    [(q0, q1)] = _spmm_sc([support2], srcg, dstg, wg)
    out = _sum_bias(q0, q1, b2p)
    return out[:, :ncls]


# speedup vs baseline: 1.6027x; 1.6027x over previous
import functools

import jax
import jax.numpy as jnp
from jax import lax
from jax.experimental import pallas as pl
from jax.experimental.pallas import tpu as pltpu
from jax.experimental.pallas import tpu_sc as plsc

N_NODES = 10000
NTILES = 32
TPSC = 16
K_EDGES = 96
NBUF = 5
ZR = 125
RPT = N_NODES // TPSC
WROWS = 632

_MBLK = 1000


def _make_mm_multi(n_parts):
    def _kern(*refs):
        x_ref = refs[0]
        ws = refs[1:1 + n_parts]
        outs = refs[1 + n_parts:]
        xv = x_ref[...]
        for w_ref, o_ref in zip(ws, outs):
            o_ref[...] = jnp.dot(xv, w_ref[...],
                                 preferred_element_type=jnp.float32)
    return _kern


def _matmul_multi(x, ws):
    m, k = x.shape
    n = ws[0].shape[1]
    return pl.pallas_call(
        _make_mm_multi(len(ws)),
        grid=(m // _MBLK,),
        in_specs=[pl.BlockSpec((_MBLK, k), lambda i: (i, 0))] +
                 [pl.BlockSpec((k, n), lambda i: (0, 0))] * len(ws),
        out_specs=[pl.BlockSpec((_MBLK, n), lambda i: (i, 0))] * len(ws),
        out_shape=[jax.ShapeDtypeStruct((m, n), jnp.float32)] * len(ws),
    )(x, *ws)


def _make_fuse(n_parts):
    def _kern(*refs):
        p0s = refs[0:n_parts]
        p1s = refs[n_parts:2 * n_parts]
        bs = refs[2 * n_parts:3 * n_parts]
        ws = refs[3 * n_parts:4 * n_parts]
        o_ref = refs[4 * n_parts]
        acc = None
        for p0, p1, b, w in zip(p0s, p1s, bs, ws):
            h = jnp.maximum(p0[...] + p1[...] + b[...], 0.0)
            t = jnp.dot(h, w[...], preferred_element_type=jnp.float32)
            acc = t if acc is None else acc + t
        o_ref[...] = acc
    return _kern


def _sum_relu_matmul(p0s, p1s, b1s, w2s):
    m, k = p0s[0].shape
    n = w2s[0].shape[1]
    np_ = len(p0s)
    blk = pl.BlockSpec((_MBLK, k), lambda i: (i, 0))
    return pl.pallas_call(
        _make_fuse(np_),
        grid=(m // _MBLK,),
        in_specs=[blk] * (2 * np_) +
                 [pl.BlockSpec((1, k), lambda i: (0, 0))] * np_ +
                 [pl.BlockSpec((k, n), lambda i: (0, 0))] * np_,
        out_specs=pl.BlockSpec((_MBLK, n), lambda i: (i, 0)),
        out_shape=jax.ShapeDtypeStruct((m, n), jnp.float32),
    )(*p0s, *p1s, *[b.reshape(1, k) for b in b1s], *w2s)


def _final_kernel(q0_ref, q1_ref, b2_ref, o_ref):
    o_ref[...] = q0_ref[...] + q1_ref[...] + b2_ref[...]


def _sum_bias(q0, q1, b2):
    m, n = q0.shape
    return pl.pallas_call(
        _final_kernel,
        grid=(m // _MBLK,),
        in_specs=[
            pl.BlockSpec((_MBLK, n), lambda i: (i, 0)),
            pl.BlockSpec((_MBLK, n), lambda i: (i, 0)),
            pl.BlockSpec((1, n), lambda i: (0, 0)),
        ],
        out_specs=pl.BlockSpec((_MBLK, n), lambda i: (i, 0)),
        out_shape=jax.ShapeDtypeStruct((m, n), jnp.float32),
    )(q0, q1, b2.reshape(1, n))


@functools.lru_cache(maxsize=None)
def _make_spmm(n_chunks: int, d: int, n_feats: int):
    mesh = plsc.VectorSubcoreMesh(core_axis_name="c", subcore_axis_name="s")
    groups = n_chunks // NBUF

    @functools.partial(
        pl.kernel,
        out_type=jax.ShapeDtypeStruct((n_feats * 2 * N_NODES, d), jnp.float32),
        mesh=mesh,
        compiler_params=pltpu.CompilerParams(use_tc_tiling_on_sc=False),
        scratch_types=(
            pltpu.VMEM((n_chunks, K_EDGES), jnp.int32),
            pltpu.VMEM((n_chunks, K_EDGES), jnp.int32),
            pltpu.VMEM((n_chunks, K_EDGES), jnp.float32),
            [pltpu.VMEM((K_EDGES, d), jnp.float32) for _ in range(NBUF)],
            pltpu.VMEM((ZR, d), jnp.float32),
            pltpu.VMEM_SHARED((N_NODES, d), jnp.float32),
            [pltpu.SemaphoreType.DMA for _ in range(NBUF)],
        ),
    )
    def spmm(*args):
        feats = args[:n_feats]
        (srcg, dstg, wg, out, srcb, dstb, wb, rows, zbuf, acc,
         gsems) = args[n_feats:]
        cid = lax.axis_index("c")
        sid = lax.axis_index("s")
        wid = cid * TPSC + sid
        r0 = sid * RPT

        pltpu.sync_copy(srcg.at[wid], srcb)
        pltpu.sync_copy(dstg.at[wid], dstb)
        pltpu.sync_copy(wg.at[wid], wb)

        nfull = d // 16
        rem = d % 16
        assert rem in (0, 8)
        lane = lax.iota(jnp.int32, 16)

        def zrow(r, carry):
            for j in range(nfull):
                zbuf[r, pl.ds(j * 16, 16)] = jnp.zeros((16,), jnp.float32)
            if rem:
                zbuf[r, pl.ds(d - 16, 16)] = jnp.zeros((16,), jnp.float32)
            return carry
        lax.fori_loop(0, ZR, zrow, 0)

        for h in range(n_feats):
            feat = feats[h]

            for i in range(RPT // ZR):
                pltpu.sync_copy(zbuf, acc.at[pl.ds(r0 + i * ZR, ZR)])
            plsc.subcore_barrier()

            for b in range(NBUF):
                pltpu.async_copy(feat.at[srcb.at[b]], rows[b], gsems[b])

            def group(g, carry):
                for b in range(NBUF):
                    c = g * NBUF + b
                    pltpu.make_async_copy(
                        feat.at[srcb.at[c]], rows[b], gsems[b]).wait()

                    def scale_grp(gi, carry2):
                        e0 = gi * 16
                        wv = wb[c, pl.ds(e0, 16)]
                        for l in range(16):
                            wsc = wv[l]
                            for j in range(nfull):
                                v = rows[b][e0 + l, pl.ds(j * 16, 16)]
                                rows[b][e0 + l, pl.ds(j * 16, 16)] = v * wsc
                            if rem:
                                wt = jnp.where(lane < 16 - rem, 1.0, wsc)
                                v = rows[b][e0 + l, pl.ds(d - 16, 16)]
                                rows[b][e0 + l, pl.ds(d - 16, 16)] = v * wt
                        return carry2
                    lax.fori_loop(0, K_EDGES // 16, scale_grp, 0)

                    pltpu.sync_copy(rows[b], acc.at[dstb.at[c]], add=True)

                    @pl.when(g < groups - 1)
                    def _refill():
                        pltpu.async_copy(
                            feat.at[srcb.at[c + NBUF]], rows[b], gsems[b])
                return carry
            lax.fori_loop(0, groups, group, 0)

            plsc.subcore_barrier()
            last = N_NODES - 15 * WROWS
            r0w = sid * WROWS
            obase = (h * 2) * N_NODES + cid * N_NODES

            @pl.when(sid < 15)
            def _wmain():
                pltpu.sync_copy(acc.at[pl.ds(r0w, WROWS)],
                                out.at[pl.ds(obase + r0w, WROWS)])

            @pl.when(sid == 15)
            def _wlast():
                pltpu.sync_copy(acc.at[pl.ds(15 * WROWS, last)],
                                out.at[pl.ds(obase + 15 * WROWS, last)])

            plsc.subcore_barrier()

    return spmm


def _spmm_sc(feats, srcg, dstg, wg):
    n_chunks = srcg.shape[1]
    d = feats[0].shape[1]
    out = _make_spmm(n_chunks, d, len(feats))(*feats, srcg, dstg, wg)
    return [(out[(2 * h) * N_NODES:(2 * h + 1) * N_NODES],
             out[(2 * h + 1) * N_NODES:(2 * h + 2) * N_NODES])
            for h in range(len(feats))]


def kernel(x, edge_index, edge_weight, W1, b1, W2, b2):
    n_edges = edge_index.shape[1]
    n_chunks = -(-n_edges // (NTILES * K_EDGES))
    n_chunks = -(-n_chunks // NBUF) * NBUF
    pad_e = NTILES * n_chunks * K_EDGES - n_edges
    srcg = jnp.pad(edge_index[0].astype(jnp.int32), (0, pad_e)).reshape(
        NTILES, n_chunks, K_EDGES)
    dstg = jnp.pad(edge_index[1].astype(jnp.int32), (0, pad_e)).reshape(
        NTILES, n_chunks, K_EDGES)
    wg = jnp.pad(edge_weight, (0, pad_e)).reshape(NTILES, n_chunks, K_EDGES)

    dslab = 48
    nslab = 3
    d2 = 48
    nfeat = W1.shape[0]
    hpad = nslab * dslab - nfeat
    ncls = W2.shape[1]
    W1p = jnp.pad(W1, ((0, 0), (0, hpad)))
    b1p = jnp.pad(b1, (0, hpad))
    W2rp = jnp.pad(W2, ((0, hpad), (0, d2 - ncls)))
    b2p = jnp.pad(b2, (0, d2 - ncls))

    w1s = [W1p[:, i * dslab:(i + 1) * dslab] for i in range(nslab)]
    b1s = [b1p[i * dslab:(i + 1) * dslab] for i in range(nslab)]
    w2s = [W2rp[i * dslab:(i + 1) * dslab] for i in range(nslab)]

    sups = _matmul_multi(x, w1s)
    parts = _spmm_sc(sups, srcg, dstg, wg)
    support2 = _sum_relu_matmul([p[0] for p in parts], [p[1] for p in parts],
                                b1s, w2s)
    [(q0, q1)] = _spmm_sc([support2], srcg, dstg, wg)
    out = _sum_bias(q0, q1, b2p)
    return out[:, :ncls]


# speedup vs baseline: 2.3862x; 1.4889x over previous
import functools

import jax
import jax.numpy as jnp
from jax import lax
from jax.experimental import pallas as pl
from jax.experimental.pallas import tpu as pltpu
from jax.experimental.pallas import tpu_sc as plsc

N_NODES = 10000
NTILES = 32
TPSC = 16
K_EDGES = 80
NBUF = 5
ZR = 125
RPT = N_NODES // TPSC
WROWS = 632

_MBLK = 1000


def _make_mm_multi(n_parts):
    def _kern(*refs):
        x_ref = refs[0]
        ws = refs[1:1 + n_parts]
        outs = refs[1 + n_parts:]
        xv = x_ref[...]
        for w_ref, o_ref in zip(ws, outs):
            o_ref[...] = jnp.dot(xv, w_ref[...],
                                 preferred_element_type=jnp.float32)
    return _kern


def _matmul_multi(x, ws):
    m, k = x.shape
    n = ws[0].shape[1]
    return pl.pallas_call(
        _make_mm_multi(len(ws)),
        grid=(m // _MBLK,),
        in_specs=[pl.BlockSpec((_MBLK, k), lambda i: (i, 0))] +
                 [pl.BlockSpec((k, n), lambda i: (0, 0))] * len(ws),
        out_specs=[pl.BlockSpec((_MBLK, n), lambda i: (i, 0))] * len(ws),
        out_shape=[jax.ShapeDtypeStruct((m, n), jnp.float32)] * len(ws),
    )(x, *ws)


def _make_fuse(n_parts):
    def _kern(*refs):
        p0s = refs[0:n_parts]
        p1s = refs[n_parts:2 * n_parts]
        bs = refs[2 * n_parts:3 * n_parts]
        ws = refs[3 * n_parts:4 * n_parts]
        o_ref = refs[4 * n_parts]
        acc = None
        for p0, p1, b, w in zip(p0s, p1s, bs, ws):
            h = jnp.maximum(p0[...] + p1[...] + b[...], 0.0)
            t = jnp.dot(h, w[...], preferred_element_type=jnp.float32)
            acc = t if acc is None else acc + t
        o_ref[...] = acc
    return _kern


def _sum_relu_matmul(p0s, p1s, b1s, w2s):
    m, k = p0s[0].shape
    n = w2s[0].shape[1]
    np_ = len(p0s)
    blk = pl.BlockSpec((_MBLK, k), lambda i: (i, 0))
    return pl.pallas_call(
        _make_fuse(np_),
        grid=(m // _MBLK,),
        in_specs=[blk] * (2 * np_) +
                 [pl.BlockSpec((1, k), lambda i: (0, 0))] * np_ +
                 [pl.BlockSpec((k, n), lambda i: (0, 0))] * np_,
        out_specs=pl.BlockSpec((_MBLK, n), lambda i: (i, 0)),
        out_shape=jax.ShapeDtypeStruct((m, n), jnp.float32),
    )(*p0s, *p1s, *[b.reshape(1, k) for b in b1s], *w2s)


def _final_kernel(q0_ref, q1_ref, b2_ref, o_ref):
    o_ref[...] = q0_ref[...] + q1_ref[...] + b2_ref[...]


def _sum_bias(q0, q1, b2):
    m, n = q0.shape
    return pl.pallas_call(
        _final_kernel,
        grid=(m // _MBLK,),
        in_specs=[
            pl.BlockSpec((_MBLK, n), lambda i: (i, 0)),
            pl.BlockSpec((_MBLK, n), lambda i: (i, 0)),
            pl.BlockSpec((1, n), lambda i: (0, 0)),
        ],
        out_specs=pl.BlockSpec((_MBLK, n), lambda i: (i, 0)),
        out_shape=jax.ShapeDtypeStruct((m, n), jnp.float32),
    )(q0, q1, b2.reshape(1, n))


@functools.lru_cache(maxsize=None)
def _make_spmm(n_chunks: int, d: int, n_feats: int):
    mesh = plsc.VectorSubcoreMesh(core_axis_name="c", subcore_axis_name="s")
    groups = n_chunks // NBUF

    @functools.partial(
        pl.kernel,
        out_type=jax.ShapeDtypeStruct((n_feats * 2 * N_NODES, d), jnp.float32),
        mesh=mesh,
        compiler_params=pltpu.CompilerParams(use_tc_tiling_on_sc=False),
        scratch_types=(
            pltpu.VMEM((n_chunks, K_EDGES), jnp.int32),
            pltpu.VMEM((n_chunks, K_EDGES), jnp.int32),
            pltpu.VMEM((n_chunks, K_EDGES), jnp.float32),
            [pltpu.VMEM((K_EDGES, d), jnp.float32) for _ in range(NBUF)],
            pltpu.VMEM((ZR, d), jnp.float32),
            pltpu.VMEM_SHARED((N_NODES, d), jnp.float32),
            [pltpu.SemaphoreType.DMA for _ in range(NBUF)],
        ),
    )
    def spmm(*args):
        feats = args[:n_feats]
        (srcg, dstg, wg, out, srcb, dstb, wb, rows, zbuf, acc,
         gsems) = args[n_feats:]
        cid = lax.axis_index("c")
        sid = lax.axis_index("s")
        wid = cid * TPSC + sid
        r0 = sid * RPT

        pltpu.sync_copy(srcg.at[wid], srcb)
        pltpu.sync_copy(dstg.at[wid], dstb)
        pltpu.sync_copy(wg.at[wid], wb)

        nfull = d // 16
        rem = d % 16
        assert rem in (0, 8)
        lane = lax.iota(jnp.int32, 16)

        def zrow(r, carry):
            for j in range(nfull):
                zbuf[r, pl.ds(j * 16, 16)] = jnp.zeros((16,), jnp.float32)
            if rem:
                zbuf[r, pl.ds(d - 16, 16)] = jnp.zeros((16,), jnp.float32)
            return carry
        lax.fori_loop(0, ZR, zrow, 0)

        for h in range(n_feats):
            feat = feats[h]

            for i in range(RPT // ZR):
                pltpu.sync_copy(zbuf, acc.at[pl.ds(r0 + i * ZR, ZR)])
            plsc.subcore_barrier()

            for b in range(NBUF):
                pltpu.async_copy(feat.at[srcb.at[b]], rows[b], gsems[b])

            def group(g, carry):
                for b in range(NBUF):
                    c = g * NBUF + b
                    pltpu.make_async_copy(
                        feat.at[srcb.at[c]], rows[b], gsems[b]).wait()

                    def scale_grp(gi, carry2):
                        e0 = gi * 16
                        wv = wb[c, pl.ds(e0, 16)]
                        for l in range(16):
                            wsc = wv[l]
                            for j in range(nfull):
                                v = rows[b][e0 + l, pl.ds(j * 16, 16)]
                                rows[b][e0 + l, pl.ds(j * 16, 16)] = v * wsc
                            if rem:
                                wt = jnp.where(lane < 16 - rem, 1.0, wsc)
                                v = rows[b][e0 + l, pl.ds(d - 16, 16)]
                                rows[b][e0 + l, pl.ds(d - 16, 16)] = v * wt
                        return carry2
                    lax.fori_loop(0, K_EDGES // 16, scale_grp, 0)

                    pltpu.sync_copy(rows[b], acc.at[dstb.at[c]], add=True)

                    @pl.when(g < groups - 1)
                    def _refill():
                        pltpu.async_copy(
                            feat.at[srcb.at[c + NBUF]], rows[b], gsems[b])
                return carry
            lax.fori_loop(0, groups, group, 0)

            plsc.subcore_barrier()
            last = N_NODES - 15 * WROWS
            r0w = sid * WROWS
            obase = (h * 2) * N_NODES + cid * N_NODES

            @pl.when(sid < 15)
            def _wmain():
                pltpu.sync_copy(acc.at[pl.ds(r0w, WROWS)],
                                out.at[pl.ds(obase + r0w, WROWS)])

            @pl.when(sid == 15)
            def _wlast():
                pltpu.sync_copy(acc.at[pl.ds(15 * WROWS, last)],
                                out.at[pl.ds(obase + 15 * WROWS, last)])

            plsc.subcore_barrier()

    return spmm


def _spmm_sc(feats, srcg, dstg, wg):
    n_chunks = srcg.shape[1]
    d = feats[0].shape[1]
    out = _make_spmm(n_chunks, d, len(feats))(*feats, srcg, dstg, wg)
    return [(out[(2 * h) * N_NODES:(2 * h + 1) * N_NODES],
             out[(2 * h + 1) * N_NODES:(2 * h + 2) * N_NODES])
            for h in range(len(feats))]


def kernel(x, edge_index, edge_weight, W1, b1, W2, b2):
    n_edges = edge_index.shape[1]
    n_chunks = -(-n_edges // (NTILES * K_EDGES))
    n_chunks = -(-n_chunks // NBUF) * NBUF
    pad_e = NTILES * n_chunks * K_EDGES - n_edges
    srcg = jnp.pad(edge_index[0].astype(jnp.int32), (0, pad_e)).reshape(
        NTILES, n_chunks, K_EDGES)
    dstg = jnp.pad(edge_index[1].astype(jnp.int32), (0, pad_e)).reshape(
        NTILES, n_chunks, K_EDGES)
    wg = jnp.pad(edge_weight, (0, pad_e)).reshape(NTILES, n_chunks, K_EDGES)

    dslab = 80
    nslab = 2
    d2 = 48
    nfeat = W1.shape[0]
    hpad = nslab * dslab - nfeat
    ncls = W2.shape[1]
    W1p = jnp.pad(W1, ((0, 0), (0, hpad)))
    b1p = jnp.pad(b1, (0, hpad))
    W2rp = jnp.pad(W2, ((0, hpad), (0, d2 - ncls)))
    b2p = jnp.pad(b2, (0, d2 - ncls))

    w1s = [W1p[:, i * dslab:(i + 1) * dslab] for i in range(nslab)]
    b1s = [b1p[i * dslab:(i + 1) * dslab] for i in range(nslab)]
    w2s = [W2rp[i * dslab:(i + 1) * dslab] for i in range(nslab)]

    sups = _matmul_multi(x, w1s)
    parts = _spmm_sc(sups, srcg, dstg, wg)
    support2 = _sum_relu_matmul([p[0] for p in parts], [p[1] for p in parts],
                                b1s, w2s)
    [(q0, q1)] = _spmm_sc([support2], srcg, dstg, wg)
    out = _sum_bias(q0, q1, b2p)
    return out[:, :ncls]


# speedup vs baseline: 2.4189x; 1.0137x over previous
import functools

import jax
import jax.numpy as jnp
from jax import lax
from jax.experimental import pallas as pl
from jax.experimental.pallas import tpu as pltpu
from jax.experimental.pallas import tpu_sc as plsc

N_NODES = 10000
NTILES = 32
TPSC = 16
K_EDGES = 80
NBUF = 5
ZR = 125
RPT = N_NODES // TPSC
WROWS = 632

_MBLK = 1000


def _make_mm_multi(n_parts):
    def _kern(*refs):
        x_ref = refs[0]
        ws = refs[1:1 + n_parts]
        outs = refs[1 + n_parts:]
        xv = x_ref[...]
        for w_ref, o_ref in zip(ws, outs):
            o_ref[...] = jnp.dot(xv, w_ref[...],
                                 preferred_element_type=jnp.float32)
    return _kern


def _matmul_multi(x, ws):
    m, k = x.shape
    n = ws[0].shape[1]
    return pl.pallas_call(
        _make_mm_multi(len(ws)),
        grid=(m // _MBLK,),
        in_specs=[pl.BlockSpec((_MBLK, k), lambda i: (i, 0))] +
                 [pl.BlockSpec((k, n), lambda i: (0, 0))] * len(ws),
        out_specs=[pl.BlockSpec((_MBLK, n), lambda i: (i, 0))] * len(ws),
        out_shape=[jax.ShapeDtypeStruct((m, n), jnp.float32)] * len(ws),
    )(x, *ws)


def _make_fuse(n_parts):
    def _kern(*refs):
        p0s = refs[0:n_parts]
        p1s = refs[n_parts:2 * n_parts]
        bs = refs[2 * n_parts:3 * n_parts]
        ws = refs[3 * n_parts:4 * n_parts]
        o_ref = refs[4 * n_parts]
        acc = None
        for p0, p1, b, w in zip(p0s, p1s, bs, ws):
            h = jnp.maximum(p0[...] + p1[...] + b[...], 0.0)
            t = jnp.dot(h, w[...], preferred_element_type=jnp.float32)
            acc = t if acc is None else acc + t
        o_ref[...] = acc
    return _kern


def _sum_relu_matmul(p0s, p1s, b1s, w2s):
    m, k = p0s[0].shape
    n = w2s[0].shape[1]
    np_ = len(p0s)
    blk = pl.BlockSpec((_MBLK, k), lambda i: (i, 0))
    return pl.pallas_call(
        _make_fuse(np_),
        grid=(m // _MBLK,),
        in_specs=[blk] * (2 * np_) +
                 [pl.BlockSpec((1, k), lambda i: (0, 0))] * np_ +
                 [pl.BlockSpec((k, n), lambda i: (0, 0))] * np_,
        out_specs=pl.BlockSpec((_MBLK, n), lambda i: (i, 0)),
        out_shape=jax.ShapeDtypeStruct((m, n), jnp.float32),
    )(*p0s, *p1s, *[b.reshape(1, k) for b in b1s], *w2s)


def _final_kernel(q0_ref, q1_ref, b2_ref, o_ref):
    o_ref[...] = q0_ref[...] + q1_ref[...] + b2_ref[...]


def _sum_bias(q0, q1, b2):
    m, n = q0.shape
    return pl.pallas_call(
        _final_kernel,
        grid=(m // _MBLK,),
        in_specs=[
            pl.BlockSpec((_MBLK, n), lambda i: (i, 0)),
            pl.BlockSpec((_MBLK, n), lambda i: (i, 0)),
            pl.BlockSpec((1, n), lambda i: (0, 0)),
        ],
        out_specs=pl.BlockSpec((_MBLK, n), lambda i: (i, 0)),
        out_shape=jax.ShapeDtypeStruct((m, n), jnp.float32),
    )(q0, q1, b2.reshape(1, n))


@functools.lru_cache(maxsize=None)
def _make_spmm(n_chunks: int, d: int, n_feats: int):
    mesh = plsc.VectorSubcoreMesh(core_axis_name="c", subcore_axis_name="s")
    groups = n_chunks // NBUF

    @functools.partial(
        pl.kernel,
        out_type=jax.ShapeDtypeStruct((n_feats * 2 * N_NODES, d), jnp.float32),
        mesh=mesh,
        compiler_params=pltpu.CompilerParams(use_tc_tiling_on_sc=False),
        scratch_types=(
            pltpu.VMEM((n_chunks, K_EDGES), jnp.int32),
            pltpu.VMEM((n_chunks, K_EDGES), jnp.int32),
            pltpu.VMEM((n_chunks, K_EDGES), jnp.float32),
            [pltpu.VMEM((K_EDGES, d), jnp.float32) for _ in range(NBUF)],
            pltpu.VMEM((ZR, d), jnp.float32),
            pltpu.VMEM_SHARED((N_NODES, d), jnp.float32),
            [pltpu.SemaphoreType.DMA for _ in range(NBUF)],
        ),
    )
    def spmm(*args):
        feats = args[:n_feats]
        (srcg, dstg, wg, out, srcb, dstb, wb, rows, zbuf, acc,
         gsems) = args[n_feats:]
        cid = lax.axis_index("c")
        sid = lax.axis_index("s")
        wid = cid * TPSC + sid
        r0 = sid * RPT

        pltpu.sync_copy(srcg.at[wid], srcb)
        pltpu.sync_copy(dstg.at[wid], dstb)
        pltpu.sync_copy(wg.at[wid], wb)

        nfull = d // 16
        rem = d % 16
        assert rem in (0, 8)
        lane = lax.iota(jnp.int32, 16)

        def zrow(r, carry):
            for j in range(nfull):
                zbuf[r, pl.ds(j * 16, 16)] = jnp.zeros((16,), jnp.float32)
            if rem:
                zbuf[r, pl.ds(d - 16, 16)] = jnp.zeros((16,), jnp.float32)
            return carry
        lax.fori_loop(0, ZR, zrow, 0)

        for h in range(n_feats):
            feat = feats[h]

            for b in range(NBUF):
                pltpu.async_copy(feat.at[srcb.at[b]], rows[b], gsems[b])

            for i in range(RPT // ZR):
                pltpu.sync_copy(zbuf, acc.at[pl.ds(r0 + i * ZR, ZR)])
            plsc.subcore_barrier()

            def group(g, carry):
                for b in range(NBUF):
                    c = g * NBUF + b
                    pltpu.make_async_copy(
                        feat.at[srcb.at[c]], rows[b], gsems[b]).wait()

                    def scale_grp(gi, carry2):
                        e0 = gi * 16
                        wv = wb[c, pl.ds(e0, 16)]
                        for l in range(16):
                            wsc = wv[l]
                            for j in range(nfull):
                                v = rows[b][e0 + l, pl.ds(j * 16, 16)]
                                rows[b][e0 + l, pl.ds(j * 16, 16)] = v * wsc
                            if rem:
                                wt = jnp.where(lane < 16 - rem, 1.0, wsc)
                                v = rows[b][e0 + l, pl.ds(d - 16, 16)]
                                rows[b][e0 + l, pl.ds(d - 16, 16)] = v * wt
                        return carry2
                    lax.fori_loop(0, K_EDGES // 16, scale_grp, 0)

                    pltpu.sync_copy(rows[b], acc.at[dstb.at[c]], add=True)

                    @pl.when(g < groups - 1)
                    def _refill():
                        pltpu.async_copy(
                            feat.at[srcb.at[c + NBUF]], rows[b], gsems[b])
                return carry
            lax.fori_loop(0, groups, group, 0)

            plsc.subcore_barrier()
            last = N_NODES - 15 * WROWS
            r0w = sid * WROWS
            obase = (h * 2) * N_NODES + cid * N_NODES

            @pl.when(sid < 15)
            def _wmain():
                pltpu.sync_copy(acc.at[pl.ds(r0w, WROWS)],
                                out.at[pl.ds(obase + r0w, WROWS)])

            @pl.when(sid == 15)
            def _wlast():
                pltpu.sync_copy(acc.at[pl.ds(15 * WROWS, last)],
                                out.at[pl.ds(obase + 15 * WROWS, last)])

            plsc.subcore_barrier()

    return spmm


def _spmm_sc(feats, srcg, dstg, wg):
    n_chunks = srcg.shape[1]
    d = feats[0].shape[1]
    out = _make_spmm(n_chunks, d, len(feats))(*feats, srcg, dstg, wg)
    return [(out[(2 * h) * N_NODES:(2 * h + 1) * N_NODES],
             out[(2 * h + 1) * N_NODES:(2 * h + 2) * N_NODES])
            for h in range(len(feats))]


def kernel(x, edge_index, edge_weight, W1, b1, W2, b2):
    n_edges = edge_index.shape[1]
    n_chunks = -(-n_edges // (NTILES * K_EDGES))
    n_chunks = -(-n_chunks // NBUF) * NBUF
    pad_e = NTILES * n_chunks * K_EDGES - n_edges
    srcg = jnp.pad(edge_index[0].astype(jnp.int32), (0, pad_e)).reshape(
        NTILES, n_chunks, K_EDGES)
    dstg = jnp.pad(edge_index[1].astype(jnp.int32), (0, pad_e)).reshape(
        NTILES, n_chunks, K_EDGES)
    wg = jnp.pad(edge_weight, (0, pad_e)).reshape(NTILES, n_chunks, K_EDGES)

    dslab = 80
    nslab = 2
    d2 = 48
    nfeat = W1.shape[0]
    hpad = nslab * dslab - nfeat
    ncls = W2.shape[1]
    W1p = jnp.pad(W1, ((0, 0), (0, hpad)))
    b1p = jnp.pad(b1, (0, hpad))
    W2rp = jnp.pad(W2, ((0, hpad), (0, d2 - ncls)))
    b2p = jnp.pad(b2, (0, d2 - ncls))

    w1s = [W1p[:, i * dslab:(i + 1) * dslab] for i in range(nslab)]
    b1s = [b1p[i * dslab:(i + 1) * dslab] for i in range(nslab)]
    w2s = [W2rp[i * dslab:(i + 1) * dslab] for i in range(nslab)]

    sups = _matmul_multi(x, w1s)
    parts = _spmm_sc(sups, srcg, dstg, wg)
    support2 = _sum_relu_matmul([p[0] for p in parts], [p[1] for p in parts],
                                b1s, w2s)
    [(q0, q1)] = _spmm_sc([support2], srcg, dstg, wg)
    out = _sum_bias(q0, q1, b2p)
    return out[:, :ncls]
